# Initial kernel scaffold; baseline (speedup 1.0000x reference)
import jax
import jax.numpy as jnp
from jax.experimental import pallas as pl


def kernel(x, edge_index, W1q, b1q, W1k, b1k, W1v, b1v, W1s, b1s, W2q, b2q, W2k, b2k, W2v, b2v, W2s, b2s, W3q, b3q, W3k, b3k, W3v, b3v, W3s, b3s):
    raise NotImplementedError("write your pallas kernel here")


# speedup vs baseline: 3.7163x; 3.7163x over previous
import functools

import jax
import jax.numpy as jnp
from jax import lax
from jax.experimental import pallas as pl
from jax.experimental.pallas import tpu as pltpu
from jax.experimental.pallas import tpu_sc as plsc

N = 10000
E = 160000
D = 256
NC = 2
NS = 16
L = 16

CH = 64
TCH = E // CH
ROWS_PER_TILE = N // NS

_mesh = plsc.VectorSubcoreMesh(core_axis_name="c", subcore_axis_name="s")
_sc_params = pltpu.CompilerParams(use_tc_tiling_on_sc=False)


def _proj_body(x_ref, w_ref, b_ref, o_ref):
    o_ref[...] = (
        jnp.dot(x_ref[...], w_ref[...], preferred_element_type=jnp.float32)
        + b_ref[...]
    )


def _tc_proj(x, wcat, bcat):
    n, d = x.shape
    do = wcat.shape[1]
    r = 1000
    return pl.pallas_call(
        _proj_body,
        grid=(n // r,),
        in_specs=[
            pl.BlockSpec((r, d), lambda i: (i, 0)),
            pl.BlockSpec((d, do), lambda i: (0, 0)),
            pl.BlockSpec((1, do), lambda i: (0, 0)),
        ],
        out_specs=pl.BlockSpec((r, do), lambda i: (i, 0)),
        out_shape=jax.ShapeDtypeStruct((n, do), jnp.float32),
    )(x, wcat, bcat.reshape(1, do))


def _merge_body2(acc_ref, skip_ref, hprev_ref, w_ref, b_ref, h_ref, p_ref, *,
                 fix_col):
    den = acc_ref[0, :, 128:129] + 1e-16
    agg = jnp.concatenate([acc_ref[0, :, :128], acc_ref[1, :, :128]], axis=1)
    t = agg / den + skip_ref[...] + hprev_ref[...]
    h = jnp.tanh(t)
    h_ref[...] = h
    p = (
        jnp.dot(h, w_ref[...], preferred_element_type=jnp.float32)
        + b_ref[...]
    )
    if fix_col is not None:
        col = lax.broadcasted_iota(jnp.int32, p.shape, 1)
        p = jnp.where(col == fix_col, 1.0, p)
    p_ref[...] = p


def _merge_body1(acc_ref, skip_ref, w_ref, b_ref, h_ref, p_ref, *, fix_col):
    den = acc_ref[0, :, 128:129] + 1e-16
    agg = jnp.concatenate([acc_ref[0, :, :128], acc_ref[1, :, :128]], axis=1)
    t = agg / den + skip_ref[...]
    h = jnp.tanh(t)
    h_ref[...] = h
    p = (
        jnp.dot(h, w_ref[...], preferred_element_type=jnp.float32)
        + b_ref[...]
    )
    if fix_col is not None:
        col = lax.broadcasted_iota(jnp.int32, p.shape, 1)
        p = jnp.where(col == fix_col, 1.0, p)
    p_ref[...] = p


def _tc_merge(acc, skip, hprev, wcat, bcat, fix_col):
    n = acc.shape[1]
    do = wcat.shape[1]
    r = 1000
    common = dict(
        grid=(n // r,),
        out_specs=[
            pl.BlockSpec((r, D), lambda i: (i, 0)),
            pl.BlockSpec((r, do), lambda i: (i, 0)),
        ],
        out_shape=[
            jax.ShapeDtypeStruct((n, D), jnp.float32),
            jax.ShapeDtypeStruct((n, do), jnp.float32),
        ],
    )
    in_specs = [
        pl.BlockSpec((2, r, 144), lambda i: (0, i, 0)),
        pl.BlockSpec((r, D), lambda i: (i, 0)),
    ]
    w_specs = [
        pl.BlockSpec((D, do), lambda i: (0, 0)),
        pl.BlockSpec((1, do), lambda i: (0, 0)),
    ]
    if hprev is None:
        body = functools.partial(_merge_body1, fix_col=fix_col)
        return pl.pallas_call(
            body, in_specs=in_specs + w_specs, **common
        )(acc, skip, wcat, bcat.reshape(1, do))
    body = functools.partial(_merge_body2, fix_col=fix_col)
    in_specs = in_specs + [pl.BlockSpec((r, D), lambda i: (i, 0))] + w_specs
    return pl.pallas_call(
        body, in_specs=in_specs, **common
    )(acc, skip, hprev, wcat, bcat.reshape(1, do))


def _final_body(acc_ref, p3_ref, z_ref, o_ref):
    agg = acc_ref[0] + acc_ref[1]
    den = agg[:, 3:4] + 1e-16
    o_ref[...] = agg / den + p3_ref[:, 48:64] + z_ref[...] * 0.1


def _tc_final(acc3, proj3, z16):
    n = acc3.shape[1]
    r = 1000
    return pl.pallas_call(
        _final_body,
        grid=(n // r,),
        in_specs=[
            pl.BlockSpec((2, r, 16), lambda i: (0, i, 0)),
            pl.BlockSpec((r, 64), lambda i: (i, 0)),
            pl.BlockSpec((r, 16), lambda i: (i, 0)),
        ],
        out_specs=pl.BlockSpec((r, 16), lambda i: (i, 0)),
        out_shape=jax.ShapeDtypeStruct((n, 16), jnp.float32),
    )(acc3, proj3, z16)


_GDN = lax.GatherDimensionNumbers(
    offset_dims=(), collapsed_slice_dims=(0,), start_index_map=(0,))


def _shuffle(v, idx):
    return lax.gather(v, idx[:, None], _GDN, slice_sizes=(1,),
                      mode=lax.GatherScatterMode.PROMISE_IN_BOUNDS)


def _lane_sum(v, lanes):
    for kk in (8, 4, 2, 1):
        v = v + _shuffle(v, lanes ^ kk)
    return v

def _sc_pass1(q, k, dst, src):
    @functools.partial(
        pl.kernel,
        out_type=jax.ShapeDtypeStruct((E,), jnp.float32),
        mesh=_mesh,
        compiler_params=_sc_params,
        scratch_types=[
            pltpu.VMEM((CH,), jnp.int32),
            pltpu.VMEM((CH,), jnp.int32),
            pltpu.VMEM((CH, D), jnp.float32),
            pltpu.VMEM((CH, D), jnp.float32),
            pltpu.VMEM((CH,), jnp.float32),
            pltpu.SemaphoreType.DMA,
            pltpu.SemaphoreType.DMA,
        ],
    )
    def kern(q_hbm, k_hbm, dst_hbm, src_hbm, ex_hbm, di, si, qb, kb, exb,
             sem1, sem2):
        c = lax.axis_index("c")
        s = lax.axis_index("s")
        w = s * NC + c
        nbase = TCH // (NC * NS)
        rem = TCH % (NC * NS)
        base = w * nbase + jnp.minimum(w, rem)
        cnt = nbase + jnp.where(w < rem, 1, 0)
        lanes = lax.broadcasted_iota(jnp.int32, (L,), 0)

        def chunk_body(t, carry):
            off = (base + t) * CH
            pltpu.sync_copy(dst_hbm.at[pl.ds(off, CH)], di)
            pltpu.sync_copy(src_hbm.at[pl.ds(off, CH)], si)
            cp1 = pltpu.async_copy(q_hbm.at[di], qb, sem1)
            cp2 = pltpu.async_copy(k_hbm.at[si], kb, sem2)
            cp1.wait()
            cp2.wait()
            for g in range(CH // L):
                exv = jnp.zeros((L,), jnp.float32)
                for j in range(L):
                    e = g * L + j
                    acc = qb[e, pl.ds(0, L)] * kb[e, pl.ds(0, L)]
                    for db in range(1, D // L):
                        acc = acc + qb[e, pl.ds(db * L, L)] * kb[e, pl.ds(db * L, L)]
                    dotv = _lane_sum(acc, lanes) * (1.0 / 16.0)
                    exv = jnp.where(lanes == j, dotv, exv)
                exb[pl.ds(g * L, L)] = jnp.exp(exv)
            pltpu.sync_copy(exb, ex_hbm.at[pl.ds(off, CH)])
            return carry

        lax.fori_loop(0, cnt, chunk_body, 0)

    return kern(q, k, dst, src)


def _sc_pass2(vlo, vhi, ex, dst, src):
    @functools.partial(
        pl.kernel,
        out_type=jax.ShapeDtypeStruct((NC, N, 144), jnp.float32),
        mesh=_mesh,
        compiler_params=_sc_params,
        scratch_types=[
            pltpu.VMEM((CH,), jnp.int32),
            pltpu.VMEM((CH,), jnp.int32),
            pltpu.VMEM((CH, 128), jnp.float32),
            pltpu.VMEM((CH,), jnp.float32),
            pltpu.VMEM((CH, 144), jnp.float32),
            pltpu.VMEM((104, 144), jnp.float32),
            pltpu.VMEM_SHARED((N, 144), jnp.float32),
            pltpu.SemaphoreType.DMA,
        ],
    )
    def kern(vlo_hbm, vhi_hbm, ex_hbm, dst_hbm, src_hbm, out_hbm,
             si, di, vb, exb, sb, zb, acc, sem):
        c = lax.axis_index("c")
        s = lax.axis_index("s")
        lanes = lax.broadcasted_iota(jnp.int32, (L,), 0)

        rbase = s * 624

        def zrow(i, carry):
            for j in range(144 // L):
                zb[i, pl.ds(j * L, L)] = jnp.zeros((L,), jnp.float32)
            return carry

        lax.fori_loop(0, 104, zrow, 0)
        for r in range(6):
            pltpu.sync_copy(zb, acc.at[pl.ds(rbase + r * 104, 104)])

        @pl.when(s == NS - 1)
        def _():
            pltpu.sync_copy(zb.at[pl.ds(0, 16)], acc.at[pl.ds(9984, 16)])

        plsc.subcore_barrier()

        nbase = TCH // NS
        rem = TCH % NS
        base = s * nbase + jnp.minimum(s, rem)
        cnt = nbase + jnp.where(s < rem, 1, 0)

        def chunk_body(t, carry):
            off = (base + t) * CH
            pltpu.sync_copy(src_hbm.at[pl.ds(off, CH)], si)
            pltpu.sync_copy(dst_hbm.at[pl.ds(off, CH)], di)
            pltpu.sync_copy(ex_hbm.at[pl.ds(off, CH)], exb)

            @pl.when(c == 0)
            def _():
                pltpu.async_copy(vlo_hbm.at[si], vb, sem).wait()

            @pl.when(c == 1)
            def _():
                pltpu.async_copy(vhi_hbm.at[si], vb, sem).wait()

            for g in range(CH // L):
                exv = exb[pl.ds(g * L, L)]
                for j in range(L):
                    e = g * L + j
                    exs = exv[j]
                    for jj in range(128 // L):
                        sb[e, pl.ds(jj * L, L)] = vb[e, pl.ds(jj * L, L)] * exs
                    sb[e, pl.ds(128, L)] = jnp.where(lanes == 0, exs, 0.0)
            pltpu.sync_copy(sb, acc.at[di], add=True)
            return carry

        lax.fori_loop(0, cnt, chunk_body, 0)
        plsc.subcore_barrier()

        for r in range(6):
            rb = rbase + r * 104
            pltpu.sync_copy(acc.at[pl.ds(rb, 104)], zb)
            pltpu.sync_copy(zb, out_hbm.at[c].at[pl.ds(rb, 104)])

        @pl.when(s == NS - 1)
        def _():
            pltpu.sync_copy(acc.at[pl.ds(9984, 16)], zb.at[pl.ds(0, 16)])
            pltpu.sync_copy(zb.at[pl.ds(0, 16)], out_hbm.at[c].at[pl.ds(9984, 16)])

    return kern(vlo, vhi, ex, dst, src)


def _sc_layer3(q3, k3, v3, dst, src):
    @functools.partial(
        pl.kernel,
        out_type=jax.ShapeDtypeStruct((NC, N, 16), jnp.float32),
        mesh=_mesh,
        compiler_params=_sc_params,
        scratch_types=[
            pltpu.VMEM((CH,), jnp.int32),
            pltpu.VMEM((CH,), jnp.int32),
            pltpu.VMEM((CH, 16), jnp.float32),
            pltpu.VMEM((CH, 16), jnp.float32),
            pltpu.VMEM((CH, 16), jnp.float32),
            pltpu.VMEM((CH,), jnp.float32),
            pltpu.VMEM((CH, 16), jnp.float32),
            pltpu.VMEM((640, 16), jnp.float32),
            pltpu.SemaphoreType.DMA,
            pltpu.SemaphoreType.DMA,
            pltpu.SemaphoreType.DMA,
            pltpu.VMEM_SHARED((N, 16), jnp.float32),
        ],
    )
    def kern(q_hbm, k_hbm, v_hbm, dst_hbm, src_hbm, out_hbm,
             di, si, qb, kb, vb, exb, sb, zb, sem1, sem2, sem3, acc):
        c = lax.axis_index("c")
        s = lax.axis_index("s")
        w = s * NC + c
        lanes = lax.broadcasted_iota(jnp.int32, (L,), 0)
        rsqrt3 = 0.5773502691896258

        rbase = s * 624

        def zrow(i, carry):
            zb[i, pl.ds(0, L)] = jnp.zeros((L,), jnp.float32)
            return carry

        lax.fori_loop(0, 640, zrow, 0)
        pltpu.sync_copy(zb.at[pl.ds(0, 624)], acc.at[pl.ds(rbase, 624)])

        @pl.when(s == NS - 1)
        def _():
            pltpu.sync_copy(zb.at[pl.ds(624, 16)], acc.at[pl.ds(9984, 16)])

        plsc.subcore_barrier()

        nbase = TCH // (NC * NS)
        rem = TCH % (NC * NS)
        base = w * nbase + jnp.minimum(w, rem)
        cnt = nbase + jnp.where(w < rem, 1, 0)

        def chunk_body(t, carry):
            off = (base + t) * CH
            pltpu.sync_copy(dst_hbm.at[pl.ds(off, CH)], di)
            pltpu.sync_copy(src_hbm.at[pl.ds(off, CH)], si)
            cp1 = pltpu.async_copy(q_hbm.at[di], qb, sem1)
            cp2 = pltpu.async_copy(k_hbm.at[si], kb, sem2)
            cp3 = pltpu.async_copy(v_hbm.at[si], vb, sem3)
            cp1.wait()
            cp2.wait()
            cp3.wait()
            for g in range(CH // L):
                exv = jnp.zeros((L,), jnp.float32)
                for j in range(L):
                    e = g * L + j
                    acc_v = qb[e, pl.ds(0, L)] * kb[e, pl.ds(0, L)]
                    dotv = _lane_sum(acc_v, lanes) * rsqrt3
                    exv = jnp.where(lanes == j, dotv, exv)
                exb[pl.ds(g * L, L)] = jnp.exp(exv)
            for g in range(CH // L):
                exv = exb[pl.ds(g * L, L)]
                for j in range(L):
                    e = g * L + j
                    sb[e, pl.ds(0, L)] = vb[e, pl.ds(0, L)] * exv[j]
            pltpu.sync_copy(sb, acc.at[di], add=True)
            return carry

        lax.fori_loop(0, cnt, chunk_body, 0)
        plsc.subcore_barrier()
        pltpu.sync_copy(acc.at[pl.ds(rbase, 624)], zb.at[pl.ds(0, 624)])
        pltpu.sync_copy(zb.at[pl.ds(0, 624)], out_hbm.at[c].at[pl.ds(rbase, 624)])

        @pl.when(s == NS - 1)
        def _():
            pltpu.sync_copy(acc.at[pl.ds(9984, 16)], zb.at[pl.ds(624, 16)])
            pltpu.sync_copy(zb.at[pl.ds(624, 16)], out_hbm.at[c].at[pl.ds(9984, 16)])

    return kern(q3, k3, v3, dst, src)


def kernel(x, edge_index, W1q, b1q, W1k, b1k, W1v, b1v, W1s, b1s,
           W2q, b2q, W2k, b2k, W2v, b2v, W2s, b2s,
           W3q, b3q, W3k, b3k, W3v, b3v, W3s, b3s):
    src = edge_index[0]
    dst = edge_index[1]

    wcat1 = jnp.concatenate([W1q, W1k, W1v, W1s], axis=1)
    bcat1 = jnp.concatenate([b1q, b1k, b1v, b1s], axis=0)
    wcat2 = jnp.concatenate([W2q, W2k, W2v, W2s], axis=1)
    bcat2 = jnp.concatenate([b2q, b2k, b2v, b2s], axis=0)

    def pad16(w, b):
        return (jnp.pad(w, ((0, 0), (0, 13))), jnp.pad(b, (0, 13)))

    w3 = [pad16(W3q, b3q), pad16(W3k, b3k), pad16(W3v, b3v), pad16(W3s, b3s)]
    wcat3 = jnp.concatenate([w for w, _ in w3], axis=1)
    bcat3 = jnp.concatenate([b for _, b in w3], axis=0)

    z = jax.random.normal(jax.random.key(42), (N, 3), dtype=jnp.float32)
    z16 = jnp.pad(z, ((0, 0), (0, 13)))

    proj1 = _tc_proj(x, wcat1, bcat1)
    q1 = proj1[:, 0:256]
    k1 = proj1[:, 256:512]
    vlo1 = proj1[:, 512:640]
    vhi1 = proj1[:, 640:768]
    skip1 = proj1[:, 768:1024]
    ex1 = _sc_pass1(q1, k1, dst, src)
    acc1 = _sc_pass2(vlo1, vhi1, ex1, dst, src)
    h1, proj2 = _tc_merge(acc1, skip1, None, wcat2, bcat2, None)

    q2 = proj2[:, 0:256]
    k2 = proj2[:, 256:512]
    vlo2 = proj2[:, 512:640]
    vhi2 = proj2[:, 640:768]
    skip2 = proj2[:, 768:1024]
    ex2 = _sc_pass1(q2, k2, dst, src)
    acc2 = _sc_pass2(vlo2, vhi2, ex2, dst, src)
    _, proj3 = _tc_merge(acc2, skip2, h1, wcat3, bcat3, 35)

    q3 = proj3[:, 0:16]
    k3 = proj3[:, 16:32]
    v3 = proj3[:, 32:48]
    acc3 = _sc_layer3(q3, k3, v3, dst, src)
    out16 = _tc_final(acc3, proj3, z16)
    return out16[:, 0:3]


# speedup vs baseline: 9.1032x; 2.4495x over previous
import functools

import jax
import jax.numpy as jnp
from jax import lax
from jax.experimental import pallas as pl
from jax.experimental.pallas import tpu as pltpu
from jax.experimental.pallas import tpu_sc as plsc

N = 10000
E = 160000
D = 256
NC = 2
NS = 16
L = 16

CH = 64
CPT1 = 80
CPT2 = 160
EP = NC * NS * CPT1 * CH

_mesh = plsc.VectorSubcoreMesh(core_axis_name="c", subcore_axis_name="s")
_sc_params = pltpu.CompilerParams(
    use_tc_tiling_on_sc=False, needs_layout_passes=False)


def _proj_body(x_ref, w_ref, b_ref, s_ref, qbf_ref, kbf_ref, vlobf_ref,
               vhibf_ref):
    p = (
        jnp.dot(x_ref[...].astype(jnp.bfloat16),
                w_ref[...].astype(jnp.bfloat16),
                preferred_element_type=jnp.float32)
        + b_ref[...]
    )
    s_ref[...] = p[:, 768:1024]
    pbf = p[:, 0:768].astype(jnp.bfloat16)
    qbf_ref[...] = pbf[:, 0:256]
    kbf_ref[...] = pbf[:, 256:512]
    vlobf_ref[...] = pbf[:, 512:640]
    vhibf_ref[...] = pbf[:, 640:768]


def _bf_outs(n, r):
    specs = [
        pl.BlockSpec((r, D), lambda i: (i, 0)),
        pl.BlockSpec((r, D), lambda i: (i, 0)),
        pl.BlockSpec((r, 128), lambda i: (i, 0)),
        pl.BlockSpec((r, 128), lambda i: (i, 0)),
    ]
    shapes = [
        jax.ShapeDtypeStruct((n, D), jnp.bfloat16),
        jax.ShapeDtypeStruct((n, D), jnp.bfloat16),
        jax.ShapeDtypeStruct((n, 128), jnp.bfloat16),
        jax.ShapeDtypeStruct((n, 128), jnp.bfloat16),
    ]
    return specs, shapes


def _tc_proj(x, wcat, bcat):
    n, d = x.shape
    do = wcat.shape[1]
    r = 2000
    bf_specs, bf_shapes = _bf_outs(n, r)
    return pl.pallas_call(
        _proj_body,
        grid=(n // r,),
        in_specs=[
            pl.BlockSpec((r, d), lambda i: (i, 0)),
            pl.BlockSpec((d, do), lambda i: (0, 0)),
            pl.BlockSpec((1, do), lambda i: (0, 0)),
        ],
        out_specs=[pl.BlockSpec((r, D), lambda i: (i, 0))] + bf_specs,
        out_shape=[jax.ShapeDtypeStruct((n, D), jnp.float32)] + bf_shapes,
    )(x, wcat, bcat.reshape(1, do))


def _merge_common(accv_ref, accd_ref, skip_ref, hprev_ref):
    den = accd_ref[0, :, 0:1] + 1e-16
    agg = jnp.concatenate([accv_ref[0], accv_ref[1]], axis=1)
    t = agg / den + skip_ref[...]
    if hprev_ref is not None:
        t = t + hprev_ref[...]
    return jnp.tanh(t)


def _merge_bodyw(accv_ref, accd_ref, skip_ref, hprev_ref, w_ref, b_ref,
                 h_ref, s_ref, qbf_ref, kbf_ref, vlobf_ref, vhibf_ref):
    h = _merge_common(accv_ref, accd_ref, skip_ref, hprev_ref)
    h_ref[...] = h
    p = (
        jnp.dot(h.astype(jnp.bfloat16), w_ref[...].astype(jnp.bfloat16),
                preferred_element_type=jnp.float32)
        + b_ref[...]
    )
    s_ref[...] = p[:, 768:1024]
    pbf = p[:, 0:768].astype(jnp.bfloat16)
    qbf_ref[...] = pbf[:, 0:256]
    kbf_ref[...] = pbf[:, 256:512]
    vlobf_ref[...] = pbf[:, 512:640]
    vhibf_ref[...] = pbf[:, 640:768]


def _merge_bodyw1(accv_ref, accd_ref, skip_ref, w_ref, b_ref, *outs):
    _merge_bodyw(accv_ref, accd_ref, skip_ref, None, w_ref, b_ref, *outs)


def _merge_body3(accv_ref, accd_ref, skip_ref, hprev_ref, w_ref, b_ref,
                 p_ref):
    h = _merge_common(accv_ref, accd_ref, skip_ref, hprev_ref)
    p = (
        jnp.dot(h.astype(jnp.bfloat16), w_ref[...].astype(jnp.bfloat16),
                preferred_element_type=jnp.float32)
        + b_ref[...]
    )
    col = lax.broadcasted_iota(jnp.int32, p.shape, 1)
    p_ref[...] = jnp.where(col == 35, 1.0, p)


def _tc_merge(acc, skip, hprev, wcat, bcat, fix_col):
    n = N
    do = wcat.shape[1]
    r = 2000
    accv, accd = acc
    in_specs = [
        pl.BlockSpec((2, r, 128), lambda i: (0, i, 0)),
        pl.BlockSpec((2, r, 16), lambda i: (0, i, 0)),
        pl.BlockSpec((r, D), lambda i: (i, 0)),
    ]
    h_spec = pl.BlockSpec((r, D), lambda i: (i, 0))
    w_specs = [
        pl.BlockSpec((D, do), lambda i: (0, 0)),
        pl.BlockSpec((1, do), lambda i: (0, 0)),
    ]
    if do == 1024:
        bf_specs, bf_shapes = _bf_outs(n, r)
        out_specs = [h_spec, pl.BlockSpec((r, D), lambda i: (i, 0))] + bf_specs
        out_shape = [
            jax.ShapeDtypeStruct((n, D), jnp.float32),
            jax.ShapeDtypeStruct((n, D), jnp.float32),
        ] + bf_shapes
        if hprev is None:
            return pl.pallas_call(
                _merge_bodyw1, grid=(n // r,), in_specs=in_specs + w_specs,
                out_specs=out_specs, out_shape=out_shape,
            )(accv, accd, skip, wcat, bcat.reshape(1, do))
        return pl.pallas_call(
            _merge_bodyw, grid=(n // r,),
            in_specs=in_specs + [h_spec] + w_specs,
            out_specs=out_specs, out_shape=out_shape,
        )(accv, accd, skip, hprev, wcat, bcat.reshape(1, do))
    return pl.pallas_call(
        _merge_body3, grid=(n // r,),
        in_specs=in_specs + [h_spec] + w_specs,
        out_specs=pl.BlockSpec((r, do), lambda i: (i, 0)),
        out_shape=jax.ShapeDtypeStruct((n, do), jnp.float32),
    )(accv, accd, skip, hprev, wcat, bcat.reshape(1, do))


def _final_body(acc_ref, p3_ref, z_ref, o_ref):
    agg = acc_ref[0] + acc_ref[1]
    den = agg[:, 3:4] + 1e-16
    o_ref[...] = agg / den + p3_ref[:, 48:64] + z_ref[...] * 0.1


def _tc_final(acc3, proj3, z16):
    n = N
    r = 2000
    return pl.pallas_call(
        _final_body,
        grid=(n // r,),
        in_specs=[
            pl.BlockSpec((2, r, 16), lambda i: (0, i, 0)),
            pl.BlockSpec((r, 64), lambda i: (i, 0)),
            pl.BlockSpec((r, 16), lambda i: (i, 0)),
        ],
        out_specs=pl.BlockSpec((r, 16), lambda i: (i, 0)),
        out_shape=jax.ShapeDtypeStruct((n, 16), jnp.float32),
    )(acc3, proj3, z16)


_GDN = lax.GatherDimensionNumbers(
    offset_dims=(), collapsed_slice_dims=(0,), start_index_map=(0,))


def _shuffle(v, idx):
    return lax.gather(v, idx[:, None], _GDN, slice_sizes=(1,),
                      mode=lax.GatherScatterMode.PROMISE_IN_BOUNDS)


def _lane_sum(v, lanes):
    for kk in (8, 4, 2, 1):
        v = v + _shuffle(v, lanes ^ kk)
    return v


def _sc_pass1(q, k, dstg2d, src2d):
    @functools.partial(
        pl.kernel,
        out_type=jax.ShapeDtypeStruct((EP,), jnp.float32),
        mesh=_mesh,
        compiler_params=_sc_params,
        scratch_types=[
            pltpu.VMEM((CPT1, CH), jnp.int32),
            pltpu.VMEM((CPT1, CH), jnp.int32),
            pltpu.VMEM((CH, D), jnp.bfloat16),
            pltpu.VMEM((CH, D), jnp.bfloat16),
            pltpu.VMEM((CH, D), jnp.bfloat16),
            pltpu.VMEM((CH, D), jnp.bfloat16),
            pltpu.VMEM((CH,), jnp.float32),
            pltpu.VMEM((CH,), jnp.float32),
            pltpu.VMEM_SHARED((N + 16, D), jnp.bfloat16),
            pltpu.SemaphoreType.DMA,
            pltpu.SemaphoreType.DMA,
            pltpu.SemaphoreType.DMA,
            pltpu.SemaphoreType.DMA,
            pltpu.SemaphoreType.DMA,
            pltpu.SemaphoreType.DMA,
        ],
    )
    def kern(q_hbm, k_hbm, dstg_hbm, src_hbm, ex_hbm, idx_d, idx_s,
             qb0, qb1, kb0, kb1, exb0, exb1, qcache,
             sq0, sq1, sk0, sk1, se0, se1):
        c = lax.axis_index("c")
        s = lax.axis_index("s")
        w = s * NC + c
        lanes = lax.broadcasted_iota(jnp.int32, (L,), 0)
        bufs = ((qb0, kb0, exb0, sq0, sk0, se0),
                (qb1, kb1, exb1, sq1, sk1, se1))

        rbase = s * 624
        pltpu.sync_copy(q_hbm.at[pl.ds(rbase, 624)],
                        qcache.at[pl.ds(rbase, 624)])

        @pl.when(s == NS - 1)
        def _():
            pltpu.sync_copy(q_hbm.at[pl.ds(9984, 16)],
                            qcache.at[pl.ds(9984, 16)])

        pltpu.sync_copy(dstg_hbm.at[pl.ds(w * CPT1, CPT1)], idx_d)
        pltpu.sync_copy(src_hbm.at[pl.ds(w * CPT1, CPT1)], idx_s)
        plsc.subcore_barrier()
        pltpu.async_copy(qcache.at[idx_d.at[0]], qb0, sq0)
        pltpu.async_copy(k_hbm.at[idx_s.at[0]], kb0, sk0)

        def pair_body(u, carry):
            for b in range(2):
                t = u * 2 + b
                qb, kb, exb, sq, sk, se = bufs[b]
                qn, kn, _, sqn, skn, _ = bufs[1 - b]

                @pl.when(t < CPT1 - 1)
                def _():
                    pltpu.async_copy(qcache.at[idx_d.at[t + 1]], qn, sqn)
                    pltpu.async_copy(k_hbm.at[idx_s.at[t + 1]], kn, skn)

                pltpu.make_async_copy(qcache.at[idx_d.at[t]], qb, sq).wait()
                pltpu.make_async_copy(k_hbm.at[idx_s.at[t]], kb, sk).wait()

                @pl.when(t >= 2)
                def _():
                    pltpu.make_async_copy(
                        exb, ex_hbm.at[pl.ds(0, CH)], se).wait()

                for g in range(CH // L):
                    exv = jnp.zeros((L,), jnp.float32)
                    for j in range(L):
                        e = g * L + j
                        acc0 = jnp.zeros((L,), jnp.float32)
                        acc1 = jnp.zeros((L,), jnp.float32)
                        for db in range(D // 32):
                            q0, q1 = plsc.unpack(
                                qb[e, pl.ds(db * 32, 32)],
                                format=plsc.PackFormat.INTERLEAVED)
                            k0, k1 = plsc.unpack(
                                kb[e, pl.ds(db * 32, 32)],
                                format=plsc.PackFormat.INTERLEAVED)
                            acc0 = acc0 + q0 * k0
                            acc1 = acc1 + q1 * k1
                        dotv = _lane_sum(acc0 + acc1, lanes) * (1.0 / 16.0)
                        exv = jnp.where(lanes == j, dotv, exv)
                    exb[pl.ds(g * L, L)] = jnp.exp(exv)
                pltpu.async_copy(
                    exb, ex_hbm.at[pl.ds((w * CPT1 + t) * CH, CH)], se)
            return carry

        lax.fori_loop(0, CPT1 // 2, pair_body, 0)
        for b in range(2):
            _, _, exb, _, _, se = bufs[b]
            pltpu.make_async_copy(exb, ex_hbm.at[pl.ds(0, CH)], se).wait()

    return kern(q, k, dstg2d, src2d)


def _sc_pass2(vlo, vhi, ex, src2d, dsts2d):
    @functools.partial(
        pl.kernel,
        out_type=(
            jax.ShapeDtypeStruct((NC, N + 16, 128), jnp.float32),
            jax.ShapeDtypeStruct((NC, N + 16, 16), jnp.float32),
        ),
        mesh=_mesh,
        compiler_params=_sc_params,
        scratch_types=[
            pltpu.VMEM((CPT2, CH), jnp.int32),
            pltpu.VMEM((2, CH), jnp.int32),
            pltpu.VMEM((2, CH), jnp.float32),
            pltpu.VMEM((CH, 128), jnp.bfloat16),
            pltpu.VMEM((CH, 128), jnp.bfloat16),
            pltpu.VMEM((CH, 128), jnp.float32),
            pltpu.VMEM((CH, 128), jnp.float32),
            pltpu.VMEM((CH, 16), jnp.float32),
            pltpu.VMEM((CH, 16), jnp.float32),
            pltpu.VMEM_SHARED((N + 16, 128), jnp.float32),
            pltpu.VMEM_SHARED((N + 16, 16), jnp.float32),
            pltpu.SemaphoreType.DMA,
            pltpu.SemaphoreType.DMA,
            pltpu.SemaphoreType.DMA,
            pltpu.SemaphoreType.DMA,
            pltpu.SemaphoreType.DMA,
            pltpu.SemaphoreType.DMA,
            pltpu.SemaphoreType.DMA,
            pltpu.SemaphoreType.DMA,
        ],
    )
    def kern(vlo_hbm, vhi_hbm, ex_hbm, srcv_hbm, dsts_hbm, outv_hbm, outd_hbm,
             idx_sc, idx_v, exc, vb0, vb1, sb0, sb1, eb0, eb1, accv, accd,
             sv0, sv1, ss0, ss1, siv0, siv1, sie0, sie1):
        c = lax.axis_index("c")
        s = lax.axis_index("s")
        lanes = lax.broadcasted_iota(jnp.int32, (L,), 0)
        rbase = s * 624
        bufs = ((vb0, sb0, eb0, sv0, ss0, siv0, sie0),
                (vb1, sb1, eb1, sv1, ss1, siv1, sie1))

        def zrow(i, carry):
            for j in range(128 // L):
                sb0[i, pl.ds(j * L, L)] = jnp.zeros((L,), jnp.float32)
            eb0[i, pl.ds(0, L)] = jnp.zeros((L,), jnp.float32)
            return carry

        lax.fori_loop(0, CH, zrow, 0)
        for r in range(9):
            pltpu.sync_copy(sb0, accv.at[pl.ds(rbase + r * CH, CH)])
            pltpu.sync_copy(eb0, accd.at[pl.ds(rbase + r * CH, CH)])
        pltpu.sync_copy(sb0.at[pl.ds(0, 48)], accv.at[pl.ds(rbase + 576, 48)])
        pltpu.sync_copy(eb0.at[pl.ds(0, 48)], accd.at[pl.ds(rbase + 576, 48)])

        @pl.when(s == NS - 1)
        def _():
            pltpu.sync_copy(sb0.at[pl.ds(0, 32)], accv.at[pl.ds(9984, 32)])
            pltpu.sync_copy(eb0.at[pl.ds(0, 32)], accd.at[pl.ds(9984, 32)])

        plsc.subcore_barrier()

        cbase = s * CPT2
        pltpu.sync_copy(dsts_hbm.at[pl.ds(cbase, CPT2)], idx_sc)
        pltpu.sync_copy(srcv_hbm.at[cbase], idx_v.at[0])
        pltpu.sync_copy(srcv_hbm.at[cbase + 1], idx_v.at[1])
        pltpu.sync_copy(ex_hbm.at[pl.ds(cbase * CH, CH)], exc.at[0])
        pltpu.sync_copy(ex_hbm.at[pl.ds((cbase + 1) * CH, CH)], exc.at[1])

        @pl.when(c == 0)
        def _():
            pltpu.async_copy(vlo_hbm.at[idx_v.at[0]], vb0, sv0)

        @pl.when(c == 1)
        def _():
            pltpu.async_copy(vhi_hbm.at[idx_v.at[0]], vb0, sv0)

        def pair_body(u, carry):
            for b in range(2):
                t = u * 2 + b
                vb, sb, eb, sv, ss, siv, sie = bufs[b]
                vn, _, _, svn, _, sivn, sien = bufs[1 - b]

                @pl.when(jnp.logical_and(t >= 1, t < CPT2 - 1))
                def _():
                    pltpu.make_async_copy(
                        srcv_hbm.at[cbase], idx_v.at[1 - b], sivn).wait()
                    pltpu.make_async_copy(
                        ex_hbm.at[pl.ds(0, CH)], exc.at[1 - b], sien).wait()

                sbp = bufs[1 - b][1]
                ebp = bufs[1 - b][2]
                ssp = bufs[1 - b][4]

                @pl.when(t >= 1)
                def _():
                    pltpu.make_async_copy(
                        sbp, accv.at[idx_sc.at[t]], ssp).wait()
                    pltpu.make_async_copy(
                        ebp, accd.at[idx_sc.at[t]], ssp).wait()

                @pl.when(jnp.logical_and(c == 0, t < CPT2 - 1))
                def _():
                    pltpu.async_copy(vlo_hbm.at[idx_v.at[1 - b]], vn, svn)

                @pl.when(jnp.logical_and(c == 1, t < CPT2 - 1))
                def _():
                    pltpu.async_copy(vhi_hbm.at[idx_v.at[1 - b]], vn, svn)

                pltpu.make_async_copy(vlo_hbm.at[idx_v.at[b]], vb, sv).wait()

                for g in range(CH // L):
                    exv = exc[b, pl.ds(g * L, L)]
                    for j in range(L):
                        e = g * L + j
                        exs = exv[j]
                        for db in range(128 // 32):
                            v0, v1 = plsc.unpack(
                                vb[e, pl.ds(db * 32, 32)],
                                format=plsc.PackFormat.INTERLEAVED)
                            sb[e, pl.ds(db * 32, L)] = v0 * exs
                            sb[e, pl.ds(db * 32 + L, L)] = v1 * exs
                        eb[e, pl.ds(0, L)] = jnp.where(lanes == 0, exs, 0.0)

                @pl.when(t < CPT2 - 2)
                def _():
                    pltpu.async_copy(
                        srcv_hbm.at[cbase + t + 2], idx_v.at[b], siv)
                    pltpu.async_copy(
                        ex_hbm.at[pl.ds((cbase + t + 2) * CH, CH)],
                        exc.at[b], sie)

                pltpu.async_copy(sb, accv.at[idx_sc.at[t]], ss, add=True)
                pltpu.async_copy(eb, accd.at[idx_sc.at[t]], ss, add=True)
            return carry

        lax.fori_loop(0, CPT2 // 2, pair_body, 0)
        pltpu.make_async_copy(sb1, accv.at[idx_sc.at[0]], ss1).wait()
        pltpu.make_async_copy(eb1, accd.at[idx_sc.at[0]], ss1).wait()
        plsc.subcore_barrier()

        for r in range(9):
            rb = rbase + r * CH
            pltpu.sync_copy(accv.at[pl.ds(rb, CH)], sb0)
            pltpu.sync_copy(sb0, outv_hbm.at[c].at[pl.ds(rb, CH)])
            pltpu.sync_copy(accd.at[pl.ds(rb, CH)], eb0)
            pltpu.sync_copy(eb0, outd_hbm.at[c].at[pl.ds(rb, CH)])
        rb = rbase + 576
        pltpu.sync_copy(accv.at[pl.ds(rb, 48)], sb0.at[pl.ds(0, 48)])
        pltpu.sync_copy(sb0.at[pl.ds(0, 48)], outv_hbm.at[c].at[pl.ds(rb, 48)])
        pltpu.sync_copy(accd.at[pl.ds(rb, 48)], eb0.at[pl.ds(0, 48)])
        pltpu.sync_copy(eb0.at[pl.ds(0, 48)], outd_hbm.at[c].at[pl.ds(rb, 48)])

        @pl.when(s == NS - 1)
        def _():
            pltpu.sync_copy(accv.at[pl.ds(9984, 16)], sb0.at[pl.ds(0, 16)])
            pltpu.sync_copy(
                sb0.at[pl.ds(0, 16)], outv_hbm.at[c].at[pl.ds(9984, 16)])
            pltpu.sync_copy(accd.at[pl.ds(9984, 16)], eb0.at[pl.ds(0, 16)])
            pltpu.sync_copy(
                eb0.at[pl.ds(0, 16)], outd_hbm.at[c].at[pl.ds(9984, 16)])

    return kern(vlo, vhi, ex, src2d, dsts2d)


def _sc_layer3(q3, k3, v3, dstg2d, src2d, dsts2d):
    @functools.partial(
        pl.kernel,
        out_type=jax.ShapeDtypeStruct((NC, N + 16, 16), jnp.float32),
        mesh=_mesh,
        compiler_params=_sc_params,
        scratch_types=[
            pltpu.VMEM((CPT1, CH), jnp.int32),
            pltpu.VMEM((CPT1, CH), jnp.int32),
            pltpu.VMEM((CPT1, CH), jnp.int32),
            pltpu.VMEM((CH, 16), jnp.float32),
            pltpu.VMEM((CH, 16), jnp.float32),
            pltpu.VMEM((CH, 16), jnp.float32),
            pltpu.VMEM((CH, 16), jnp.float32),
            pltpu.VMEM((CH, 16), jnp.float32),
            pltpu.VMEM((CH, 16), jnp.float32),
            pltpu.VMEM((CH, 16), jnp.float32),
            pltpu.VMEM((CH, 16), jnp.float32),
            pltpu.VMEM((640, 16), jnp.float32),
            pltpu.VMEM_SHARED((N + 16, 16), jnp.float32),
            pltpu.SemaphoreType.DMA,
            pltpu.SemaphoreType.DMA,
            pltpu.SemaphoreType.DMA,
            pltpu.SemaphoreType.DMA,
            pltpu.SemaphoreType.DMA,
            pltpu.SemaphoreType.DMA,
            pltpu.SemaphoreType.DMA,
            pltpu.SemaphoreType.DMA,
        ],
    )
    def kern(q_hbm, k_hbm, v_hbm, dstg_hbm, src_hbm, dsts_hbm, out_hbm,
             idx_d, idx_s, idx_sc, qb0, qb1, kb0, kb1, vb0, vb1, sb0, sb1,
             zb, acc, sq0, sq1, sk0, sk1, sv0, sv1, ss0, ss1):
        c = lax.axis_index("c")
        s = lax.axis_index("s")
        w = s * NC + c
        lanes = lax.broadcasted_iota(jnp.int32, (L,), 0)
        rsqrt3 = 0.5773502691896258
        rbase = s * 624
        bufs = ((qb0, kb0, vb0, sb0, sq0, sk0, sv0, ss0),
                (qb1, kb1, vb1, sb1, sq1, sk1, sv1, ss1))

        def zrow(i, carry):
            zb[i, pl.ds(0, L)] = jnp.zeros((L,), jnp.float32)
            return carry

        lax.fori_loop(0, 640, zrow, 0)
        pltpu.sync_copy(zb.at[pl.ds(0, 624)], acc.at[pl.ds(rbase, 624)])

        @pl.when(s == NS - 1)
        def _():
            pltpu.sync_copy(zb.at[pl.ds(0, 32)], acc.at[pl.ds(9984, 32)])

        plsc.subcore_barrier()

        pltpu.sync_copy(dstg_hbm.at[pl.ds(w * CPT1, CPT1)], idx_d)
        pltpu.sync_copy(src_hbm.at[pl.ds(w * CPT1, CPT1)], idx_s)
        pltpu.sync_copy(dsts_hbm.at[pl.ds(w * CPT1, CPT1)], idx_sc)
        pltpu.async_copy(q_hbm.at[idx_d.at[0]], qb0, sq0)
        pltpu.async_copy(k_hbm.at[idx_s.at[0]], kb0, sk0)
        pltpu.async_copy(v_hbm.at[idx_s.at[0]], vb0, sv0)

        def pair_body(u, carry):
            for b in range(2):
                t = u * 2 + b
                qb, kb, vb, sb, sq, sk, sv, ss = bufs[b]
                qn, kn, vn, _, sqn, skn, svn, _ = bufs[1 - b]

                @pl.when(t < CPT1 - 1)
                def _():
                    pltpu.async_copy(q_hbm.at[idx_d.at[t + 1]], qn, sqn)
                    pltpu.async_copy(k_hbm.at[idx_s.at[t + 1]], kn, skn)
                    pltpu.async_copy(v_hbm.at[idx_s.at[t + 1]], vn, svn)

                pltpu.make_async_copy(q_hbm.at[idx_d.at[t]], qb, sq).wait()
                pltpu.make_async_copy(k_hbm.at[idx_s.at[t]], kb, sk).wait()
                pltpu.make_async_copy(v_hbm.at[idx_s.at[t]], vb, sv).wait()

                sbp = bufs[1 - b][3]
                ssp = bufs[1 - b][7]

                @pl.when(t >= 1)
                def _():
                    pltpu.make_async_copy(sbp, acc.at[idx_sc.at[t]], ssp).wait()

                for g in range(CH // L):
                    exv = jnp.zeros((L,), jnp.float32)
                    for j in range(L):
                        e = g * L + j
                        acc_v = qb[e, pl.ds(0, L)] * kb[e, pl.ds(0, L)]
                        dotv = _lane_sum(acc_v, lanes) * rsqrt3
                        exv = jnp.where(lanes == j, dotv, exv)
                    exvv = jnp.exp(exv)
                    for j in range(L):
                        e = g * L + j
                        sb[e, pl.ds(0, L)] = vb[e, pl.ds(0, L)] * exvv[j]
                pltpu.async_copy(sb, acc.at[idx_sc.at[t]], ss, add=True)
            return carry

        lax.fori_loop(0, CPT1 // 2, pair_body, 0)
        pltpu.make_async_copy(sb1, acc.at[idx_sc.at[0]], ss1).wait()
        plsc.subcore_barrier()
        pltpu.sync_copy(acc.at[pl.ds(rbase, 624)], zb.at[pl.ds(0, 624)])
        pltpu.sync_copy(
            zb.at[pl.ds(0, 624)], out_hbm.at[c].at[pl.ds(rbase, 624)])

        @pl.when(s == NS - 1)
        def _():
            pltpu.sync_copy(acc.at[pl.ds(9984, 16)], zb.at[pl.ds(624, 16)])
            pltpu.sync_copy(
                zb.at[pl.ds(624, 16)], out_hbm.at[c].at[pl.ds(9984, 16)])

    return kern(q3, k3, v3, dstg2d, src2d, dsts2d)


def kernel(x, edge_index, W1q, b1q, W1k, b1k, W1v, b1v, W1s, b1s,
           W2q, b2q, W2k, b2k, W2v, b2v, W2s, b2s,
           W3q, b3q, W3k, b3k, W3v, b3v, W3s, b3s):
    src = edge_index[0]
    dst = edge_index[1]
    pade = EP - E
    src2d = jnp.pad(src, (0, pade)).reshape(EP // CH, CH)
    dstg2d = jnp.pad(dst, (0, pade)).reshape(EP // CH, CH)
    dsts2d = jnp.pad(dst, (0, pade), constant_values=N).reshape(EP // CH, CH)

    def permv(wm):
        return wm.reshape(D, 8, 2, 16).swapaxes(2, 3).reshape(D, 256)

    def permvb(bm):
        return bm.reshape(8, 2, 16).swapaxes(1, 2).reshape(256)

    wcat1 = jnp.concatenate([W1q, W1k, permv(W1v), W1s], axis=1)
    bcat1 = jnp.concatenate([b1q, b1k, permvb(b1v), b1s], axis=0)
    wcat2 = jnp.concatenate([W2q, W2k, permv(W2v), W2s], axis=1)
    bcat2 = jnp.concatenate([b2q, b2k, permvb(b2v), b2s], axis=0)

    def pad16(wm, bm):
        return (jnp.pad(wm, ((0, 0), (0, 13))), jnp.pad(bm, (0, 13)))

    w3 = [pad16(W3q, b3q), pad16(W3k, b3k), pad16(W3v, b3v), pad16(W3s, b3s)]
    wcat3 = jnp.concatenate([wm for wm, _ in w3], axis=1)
    bcat3 = jnp.concatenate([bm for _, bm in w3], axis=0)

    z = jax.random.normal(jax.random.key(42), (N, 3), dtype=jnp.float32)
    z16 = jnp.pad(z, ((0, 0), (0, 13)))

    skip1, q1, k1, vlo1, vhi1 = _tc_proj(x, wcat1, bcat1)
    ex1 = _sc_pass1(q1, k1, dstg2d, src2d)
    acc1 = _sc_pass2(vlo1, vhi1, ex1, src2d, dsts2d)
    h1, skip2, q2, k2, vlo2, vhi2 = _tc_merge(
        acc1, skip1, None, wcat2, bcat2, None)

    ex2 = _sc_pass1(q2, k2, dstg2d, src2d)
    acc2 = _sc_pass2(vlo2, vhi2, ex2, src2d, dsts2d)
    proj3 = _tc_merge(acc2, skip2, h1, wcat3, bcat3, 35)

    q3 = proj3[:, 0:16]
    k3 = proj3[:, 16:32]
    v3 = proj3[:, 32:48]
    acc3 = _sc_layer3(q3, k3, v3, dstg2d, src2d, dsts2d)
    out16 = _tc_final(acc3, proj3, z16)
    return out16[:, 0:3]
